# baseline (speedup 1.0000x reference)
import functools

import jax
import jax.numpy as jnp
from jax import lax
from jax.experimental import pallas as pl
from jax.experimental.pallas import tpu as pltpu
from jax.experimental.pallas import tpu_sc as plsc

N_TABLE = 100000
EMBED_DIM = 64
NUM_BAGS = 4096
TOTAL_INDICES = 204800

NC = 2
NS = 16
NW = NC * NS
CHUNK = 640
SCAT = 128
N_BATCH = CHUNK // SCAT
PER_W = TOTAL_INDICES // NW
N_CHUNKS = PER_W // CHUNK
LOG2_BAGS = 12


def _sc_partials():
    mesh = plsc.VectorSubcoreMesh(core_axis_name="c", subcore_axis_name="s")

    @functools.partial(
        pl.kernel,
        mesh=mesh,
        compiler_params=pltpu.CompilerParams(
            needs_layout_passes=False, use_tc_tiling_on_sc=False),
        out_type=jax.ShapeDtypeStruct((NC * NUM_BAGS, EMBED_DIM), jnp.float32),
        scratch_types=[
            pltpu.VMEM((NUM_BAGS,), jnp.int32),
            pltpu.VMEM((PER_W,), jnp.int32),
            pltpu.VMEM((SCAT,), jnp.int32),
            pltpu.VMEM((SCAT,), jnp.int32),
            pltpu.VMEM((SCAT,), jnp.int32),
            pltpu.VMEM((SCAT,), jnp.int32),
            pltpu.VMEM((SCAT,), jnp.int32),
            pltpu.VMEM((SCAT,), jnp.int32),
            pltpu.VMEM((SCAT,), jnp.int32),
            pltpu.VMEM((SCAT,), jnp.int32),
            pltpu.VMEM((SCAT,), jnp.int32),
            pltpu.VMEM((SCAT,), jnp.int32),
            pltpu.VMEM((CHUNK, EMBED_DIM), jnp.float32),
            pltpu.VMEM((CHUNK, EMBED_DIM), jnp.float32),
            pltpu.VMEM_SHARED((NUM_BAGS, EMBED_DIM), jnp.float32),
            pltpu.SemaphoreType.DMA,
            pltpu.SemaphoreType.DMA,
            pltpu.SemaphoreType.DMA,
            pltpu.SemaphoreType.DMA,
        ],
    )
    def body(weight_hbm, idx_hbm, offs_hbm, out_hbm, offs_v, idx_all,
             s0a, s0b, s0c, s0d, s0e, s1a, s1b, s1c, s1d, s1e,
             rows0, rows1, acc_sh, sem_g0, sem_g1, sem_s0, sem_s1):
        c = lax.axis_index("c")
        s = lax.axis_index("s")
        wid = s * NC + c

        pltpu.sync_copy(offs_hbm, offs_v)
        pltpu.sync_copy(idx_hbm.at[pl.ds(wid * PER_W, PER_W)], idx_all)

        zeros16 = jnp.zeros((16,), jnp.float32)
        rows_per_tile = NUM_BAGS // NS

        def zero_row(r, _):
            for d in range(EMBED_DIM // 16):
                rows0[r, pl.ds(d * 16, 16)] = zeros16
            return _

        lax.fori_loop(0, rows_per_tile, zero_row, None)
        pltpu.sync_copy(rows0.at[pl.ds(0, rows_per_tile)],
                        acc_sh.at[pl.ds(s * rows_per_tile, rows_per_tile)])
        plsc.subcore_barrier()

        bufs = (((s0a, s0b, s0c, s0d, s0e), rows0, sem_g0, sem_s0),
                ((s1a, s1b, s1c, s1d, s1e), rows1, sem_g1, sem_s1))

        iota16 = lax.broadcasted_iota(jnp.int32, (16,), 0)

        def offs_at(k):
            kc = jnp.minimum(k, NUM_BAGS - 1)
            return lax.reduce_max(
                plsc.load_gather(offs_v, [jnp.full((16,), kc, jnp.int32)]),
                (0,))

        def batch_segs(base, s0, seg_ref):
            max_pos = base + SCAT - 1

            def w_cond(carry):
                k, vk = carry[0], carry[1]
                return (k < NUM_BAGS) & (vk <= max_pos)

            def w_body(carry):
                k, vk = carry[0], carry[1]
                cs = carry[2:]
                vkv = jnp.full((16,), vk, jnp.int32)
                cs = tuple(
                    cs[v] + jnp.where(vkv <= base + v * 16 + iota16, 1, 0)
                    for v in range(SCAT // 16))
                return (k + 1, offs_at(k + 1)) + cs

            init = (s0 + 1, offs_at(s0 + 1)) + tuple(
                jnp.zeros((16,), jnp.int32) for _ in range(SCAT // 16))
            out = lax.while_loop(w_cond, w_body, init)
            cs = out[2:]
            last = None
            for v in range(SCAT // 16):
                seg = jnp.minimum(s0 + cs[v], NUM_BAGS - 1)
                seg_ref[pl.ds(v * 16, 16)] = seg
                last = seg
            return lax.reduce_max(last, (0,))

        def compute_segs(g, seg_refs):
            base = wid * PER_W + g * CHUNK
            pos0 = base + iota16
            lo = jnp.zeros((16,), jnp.int32)
            hi = jnp.full((16,), NUM_BAGS, jnp.int32)
            for _step in range(LOG2_BAGS):
                mid = (lo + hi) >> 1
                val = plsc.load_gather(offs_v, [mid])
                go_right = val <= pos0
                lo = jnp.where(go_right, mid + 1, lo)
                hi = jnp.where(go_right, hi, mid)
            s0 = jnp.maximum(lax.reduce_min(lo - 1, (0,)), 0)
            for j in range(N_BATCH):
                s0 = batch_segs(base + j * SCAT, s0, seg_refs[j])

        def idx_slice(g):
            return idx_all.at[pl.ds(g * CHUNK, CHUNK)]

        def fire_gather(g, b):
            _, rows, semg, _ = bufs[b]
            pltpu.async_copy(weight_hbm.at[idx_slice(g)], rows, semg)

        def wait_gather(g, b):
            _, rows, semg, _ = bufs[b]
            pltpu.make_async_copy(
                weight_hbm.at[idx_slice(g)], rows, semg).wait()

        def fire_scatter(b):
            segs, rows, _, sems = bufs[b]
            for j in range(N_BATCH):
                pltpu.async_copy(rows.at[pl.ds(j * SCAT, SCAT)],
                                 acc_sh.at[segs[j]], sems, add=True)

        def wait_scatter(b):
            segs, rows, _, sems = bufs[b]
            for j in range(N_BATCH):
                pltpu.make_async_copy(rows.at[pl.ds(j * SCAT, SCAT)],
                                      acc_sh.at[segs[j]], sems).wait()

        fire_gather(0, 0)
        for g in range(N_CHUNKS):
            b = g % 2
            wait_gather(g, b)
            compute_segs(g, bufs[b][0])
            if g >= 1:
                wait_scatter(1 - b)
            if g + 1 < N_CHUNKS:
                fire_gather(g + 1, 1 - b)
            fire_scatter(b)
        wait_scatter((N_CHUNKS - 1) % 2)
        plsc.subcore_barrier()

        pltpu.sync_copy(
            acc_sh.at[pl.ds(s * rows_per_tile, rows_per_tile)],
            out_hbm.at[pl.ds(c * NUM_BAGS + s * rows_per_tile,
                             rows_per_tile)])

    return body


def _combine_body(a_ref, b_ref, o_ref):
    o_ref[...] = a_ref[...] + b_ref[...]


@jax.jit
def kernel(weight, indices_fwd, offsets):
    partials = _sc_partials()(weight, indices_fwd, offsets)
    return pl.pallas_call(
        _combine_body,
        out_shape=jax.ShapeDtypeStruct((NUM_BAGS, EMBED_DIM), jnp.float32),
    )(partials[:NUM_BAGS], partials[NUM_BAGS:])


# speedup vs baseline: 1.0560x; 1.0560x over previous
import functools

import jax
import jax.numpy as jnp
from jax import lax
from jax.experimental import pallas as pl
from jax.experimental.pallas import tpu as pltpu
from jax.experimental.pallas import tpu_sc as plsc

N_TABLE = 100000
EMBED_DIM = 64
NUM_BAGS = 4096
TOTAL_INDICES = 204800

NC = 2
NS = 16
NW = NC * NS
CHUNK = 256
SCAT = 128
NBUF = 3
PER_W = TOTAL_INDICES // NW
N_CHUNKS = PER_W // CHUNK
LOG2_BAGS = 12


def _sc_partials():
    mesh = plsc.VectorSubcoreMesh(core_axis_name="c", subcore_axis_name="s")

    @functools.partial(
        pl.kernel,
        mesh=mesh,
        compiler_params=pltpu.CompilerParams(
            needs_layout_passes=False, use_tc_tiling_on_sc=False),
        out_type=jax.ShapeDtypeStruct((NC * NUM_BAGS, EMBED_DIM), jnp.float32),
        scratch_types=[
            pltpu.VMEM((NUM_BAGS,), jnp.int32),
            pltpu.VMEM((PER_W,), jnp.int32),
            pltpu.VMEM((SCAT,), jnp.int32),
            pltpu.VMEM((SCAT,), jnp.int32),
            pltpu.VMEM((SCAT,), jnp.int32),
            pltpu.VMEM((SCAT,), jnp.int32),
            pltpu.VMEM((SCAT,), jnp.int32),
            pltpu.VMEM((SCAT,), jnp.int32),
            pltpu.VMEM((CHUNK, EMBED_DIM), jnp.float32),
            pltpu.VMEM((CHUNK, EMBED_DIM), jnp.float32),
            pltpu.VMEM((CHUNK, EMBED_DIM), jnp.float32),
            pltpu.VMEM_SHARED((NUM_BAGS, EMBED_DIM), jnp.float32),
            pltpu.SemaphoreType.DMA,
            pltpu.SemaphoreType.DMA,
            pltpu.SemaphoreType.DMA,
            pltpu.SemaphoreType.DMA,
            pltpu.SemaphoreType.DMA,
            pltpu.SemaphoreType.DMA,
        ],
    )
    def body(weight_hbm, idx_hbm, offs_hbm, out_hbm, offs_v, idx_all,
             seg0a, seg0b, seg1a, seg1b, seg2a, seg2b, rows0, rows1, rows2,
             acc_sh, sem_g0, sem_g1, sem_g2, sem_s0, sem_s1, sem_s2):
        c = lax.axis_index("c")
        s = lax.axis_index("s")
        wid = s * NC + c

        pltpu.sync_copy(idx_hbm.at[pl.ds(wid * PER_W, PER_W)], idx_all)
        pltpu.sync_copy(offs_hbm, offs_v)

        bufs = (((seg0a, seg0b), rows0, sem_g0, sem_s0),
                ((seg1a, seg1b), rows1, sem_g1, sem_s1),
                ((seg2a, seg2b), rows2, sem_g2, sem_s2))

        def idx_slice(g):
            return idx_all.at[pl.ds(g * CHUNK, CHUNK)]

        def fire_gather(g, b):
            _, rows, semg, _ = bufs[b]
            pltpu.async_copy(weight_hbm.at[idx_slice(g)], rows, semg)

        def wait_gather(g, b):
            _, rows, semg, _ = bufs[b]
            pltpu.make_async_copy(
                weight_hbm.at[idx_slice(g)], rows, semg).wait()

        def fire_scatter(b):
            segs, rows, _, sems = bufs[b]
            pltpu.async_copy(
                rows.at[pl.ds(0, SCAT)], acc_sh.at[segs[0]], sems, add=True)
            pltpu.async_copy(
                rows.at[pl.ds(SCAT, SCAT)], acc_sh.at[segs[1]], sems, add=True)

        def wait_scatter(b):
            segs, rows, _, sems = bufs[b]
            pltpu.make_async_copy(
                rows.at[pl.ds(0, SCAT)], acc_sh.at[segs[0]], sems).wait()
            pltpu.make_async_copy(
                rows.at[pl.ds(SCAT, SCAT)], acc_sh.at[segs[1]], sems).wait()

        fire_gather(0, 0)

        zeros16 = jnp.zeros((16,), jnp.float32)
        rows_per_tile = NUM_BAGS // NS

        def zero_row(r, _):
            for d in range(EMBED_DIM // 16):
                rows1[r, pl.ds(d * 16, 16)] = zeros16
            return _

        lax.fori_loop(0, rows_per_tile, zero_row, None)
        pltpu.sync_copy(rows1.at[pl.ds(0, rows_per_tile)],
                        acc_sh.at[pl.ds(s * rows_per_tile, rows_per_tile)])
        plsc.subcore_barrier()

        iota16 = lax.broadcasted_iota(jnp.int32, (16,), 0)
        n_vecs = CHUNK // 16

        def compute_segs(g, seg_refs):
            base = wid * PER_W + g * CHUNK
            pos0 = base + iota16
            lo = jnp.zeros((16,), jnp.int32)
            hi = jnp.full((16,), NUM_BAGS, jnp.int32)
            for _step in range(LOG2_BAGS):
                mid = (lo + hi) >> 1
                val = plsc.load_gather(offs_v, [mid])
                go_right = val <= pos0
                lo = jnp.where(go_right, mid + 1, lo)
                hi = jnp.where(go_right, hi, mid)
            s0 = jnp.maximum(lax.reduce_min(lo - 1, (0,)), 0)
            max_pos = base + CHUNK - 1

            def offs_at(k):
                kc = jnp.minimum(k, NUM_BAGS - 1)
                return lax.reduce_max(
                    plsc.load_gather(offs_v, [jnp.full((16,), kc, jnp.int32)]),
                    (0,))

            def w_cond(carry):
                k, vk = carry[0], carry[1]
                return (k < NUM_BAGS) & (vk <= max_pos)

            def w_body(carry):
                k, vk = carry[0], carry[1]
                cs = carry[2:]
                vkv = jnp.full((16,), vk, jnp.int32)
                cs = tuple(
                    cs[v] + jnp.where(vkv <= base + v * 16 + iota16, 1, 0)
                    for v in range(n_vecs))
                return (k + 1, offs_at(k + 1)) + cs

            init = (s0 + 1, offs_at(s0 + 1)) + tuple(
                jnp.zeros((16,), jnp.int32) for _ in range(n_vecs))
            out = lax.while_loop(w_cond, w_body, init)
            cs = out[2:]
            for v in range(n_vecs):
                seg = jnp.minimum(s0 + cs[v], NUM_BAGS - 1)
                half, off = divmod(v, SCAT // 16)
                seg_refs[half][pl.ds(off * 16, 16)] = seg

        def stage(g, b, fire_next, wait_old):
            bn = (b + 1) % NBUF
            wait_gather(g, b)
            if wait_old:
                wait_scatter(bn)
            if fire_next:
                fire_gather(g + 1, bn)
            compute_segs(g, bufs[b][0])
            fire_scatter(b)

        stage(0, 0, True, False)
        stage(1, 1, True, False)

        def triple(i, _):
            g = 3 * i + 2
            stage(g, 2, True, True)
            stage(g + 1, 0, True, True)
            stage(g + 2, 1, True, True)
            return _

        lax.fori_loop(0, (N_CHUNKS - 4) // 3, triple, None)
        stage(N_CHUNKS - 2, (N_CHUNKS - 2) % NBUF, True, True)
        stage(N_CHUNKS - 1, (N_CHUNKS - 1) % NBUF, False, True)
        wait_scatter((N_CHUNKS - 2) % NBUF)
        wait_scatter((N_CHUNKS - 1) % NBUF)
        plsc.subcore_barrier()

        pltpu.sync_copy(
            acc_sh.at[pl.ds(s * rows_per_tile, rows_per_tile)],
            out_hbm.at[pl.ds(c * NUM_BAGS + s * rows_per_tile,
                             rows_per_tile)])

    return body


def _combine_body(a_ref, b_ref, o_ref):
    o_ref[...] = a_ref[...] + b_ref[...]


@jax.jit
def kernel(weight, indices_fwd, offsets):
    partials = _sc_partials()(weight, indices_fwd, offsets)
    return pl.pallas_call(
        _combine_body,
        out_shape=jax.ShapeDtypeStruct((NUM_BAGS, EMBED_DIM), jnp.float32),
    )(partials[:NUM_BAGS], partials[NUM_BAGS:])


# speedup vs baseline: 1.0578x; 1.0017x over previous
import functools

import jax
import jax.numpy as jnp
from jax import lax
from jax.experimental import pallas as pl
from jax.experimental.pallas import tpu as pltpu
from jax.experimental.pallas import tpu_sc as plsc

N_TABLE = 100000
EMBED_DIM = 64
NUM_BAGS = 4096
TOTAL_INDICES = 204800

NC = 2
NS = 16
NW = NC * NS
CHUNK = 256
SCAT = 128
NBUF = 3
PER_W = TOTAL_INDICES // NW
N_CHUNKS = PER_W // CHUNK
LOG2_BAGS = 12


def _sc_partials():
    mesh = plsc.VectorSubcoreMesh(core_axis_name="c", subcore_axis_name="s")

    @functools.partial(
        pl.kernel,
        mesh=mesh,
        compiler_params=pltpu.CompilerParams(
            needs_layout_passes=False, use_tc_tiling_on_sc=False),
        out_type=jax.ShapeDtypeStruct((NC * NUM_BAGS, EMBED_DIM), jnp.float32),
        scratch_types=[
            pltpu.VMEM((NUM_BAGS,), jnp.int32),
            pltpu.VMEM((PER_W,), jnp.int32),
            pltpu.VMEM((SCAT,), jnp.int32),
            pltpu.VMEM((SCAT,), jnp.int32),
            pltpu.VMEM((SCAT,), jnp.int32),
            pltpu.VMEM((SCAT,), jnp.int32),
            pltpu.VMEM((SCAT,), jnp.int32),
            pltpu.VMEM((SCAT,), jnp.int32),
            pltpu.VMEM((CHUNK, EMBED_DIM), jnp.float32),
            pltpu.VMEM((CHUNK, EMBED_DIM), jnp.float32),
            pltpu.VMEM((CHUNK, EMBED_DIM), jnp.float32),
            pltpu.VMEM_SHARED((NUM_BAGS, EMBED_DIM), jnp.float32),
            pltpu.SemaphoreType.DMA,
            pltpu.SemaphoreType.DMA,
            pltpu.SemaphoreType.DMA,
            pltpu.SemaphoreType.DMA,
            pltpu.SemaphoreType.DMA,
            pltpu.SemaphoreType.DMA,
        ],
    )
    def body(weight_hbm, idx_hbm, offs_hbm, out_hbm, offs_v, idx_all,
             seg0a, seg0b, seg1a, seg1b, seg2a, seg2b, rows0, rows1, rows2,
             acc_sh, sem_g0, sem_g1, sem_g2, sem_s0, sem_s1, sem_s2):
        c = lax.axis_index("c")
        s = lax.axis_index("s")
        wid = s * NC + c

        pltpu.sync_copy(idx_hbm.at[pl.ds(wid * PER_W, PER_W)], idx_all)
        pltpu.sync_copy(offs_hbm, offs_v)

        bufs = (((seg0a, seg0b), rows0, sem_g0, sem_s0),
                ((seg1a, seg1b), rows1, sem_g1, sem_s1),
                ((seg2a, seg2b), rows2, sem_g2, sem_s2))

        def idx_slice(g):
            return idx_all.at[pl.ds(g * CHUNK, CHUNK)]

        def fire_gather(g, b):
            _, rows, semg, _ = bufs[b]
            pltpu.async_copy(weight_hbm.at[idx_slice(g)], rows, semg)

        def wait_gather(g, b):
            _, rows, semg, _ = bufs[b]
            pltpu.make_async_copy(
                weight_hbm.at[idx_slice(g)], rows, semg).wait()

        def fire_scatter(b):
            segs, rows, _, sems = bufs[b]
            pltpu.async_copy(
                rows.at[pl.ds(0, SCAT)], acc_sh.at[segs[0]], sems, add=True)
            pltpu.async_copy(
                rows.at[pl.ds(SCAT, SCAT)], acc_sh.at[segs[1]], sems, add=True)

        def wait_scatter(b):
            segs, rows, _, sems = bufs[b]
            pltpu.make_async_copy(
                rows.at[pl.ds(0, SCAT)], acc_sh.at[segs[0]], sems).wait()
            pltpu.make_async_copy(
                rows.at[pl.ds(SCAT, SCAT)], acc_sh.at[segs[1]], sems).wait()

        fire_gather(0, 0)

        zeros16 = jnp.zeros((16,), jnp.float32)
        rows_per_tile = NUM_BAGS // NS

        def zero_row(r, _):
            for d in range(EMBED_DIM // 16):
                rows1[r, pl.ds(d * 16, 16)] = zeros16
            return _

        lax.fori_loop(0, rows_per_tile, zero_row, None)
        pltpu.sync_copy(rows1.at[pl.ds(0, rows_per_tile)],
                        acc_sh.at[pl.ds(s * rows_per_tile, rows_per_tile)])
        plsc.subcore_barrier()

        iota16 = lax.broadcasted_iota(jnp.int32, (16,), 0)
        n_vecs = CHUNK // 16

        def compute_segs(g, seg_refs):
            base = wid * PER_W + g * CHUNK
            pos0 = base + iota16
            lo = jnp.zeros((16,), jnp.int32)
            hi = jnp.full((16,), NUM_BAGS, jnp.int32)
            for _step in range(LOG2_BAGS):
                mid = (lo + hi) >> 1
                val = plsc.load_gather(offs_v, [mid])
                go_right = val <= pos0
                lo = jnp.where(go_right, mid + 1, lo)
                hi = jnp.where(go_right, hi, mid)
            s0 = jnp.maximum(lo[0] - 1, 0)
            max_pos = base + CHUNK - 1

            def offs_at(k):
                kc = jnp.minimum(k, NUM_BAGS - 1)
                return plsc.load_gather(
                    offs_v, [jnp.full((16,), kc, jnp.int32)])[0]

            def w_cond(carry):
                k, vk = carry[0], carry[1]
                return (k < NUM_BAGS) & (vk <= max_pos)

            def w_body(carry):
                k, vk = carry[0], carry[1]
                cs = carry[2:]
                vkv = jnp.full((16,), vk, jnp.int32)
                cs = tuple(
                    cs[v] + jnp.where(vkv <= base + v * 16 + iota16, 1, 0)
                    for v in range(n_vecs))
                return (k + 1, offs_at(k + 1)) + cs

            init = (s0 + 1, offs_at(s0 + 1)) + tuple(
                jnp.zeros((16,), jnp.int32) for _ in range(n_vecs))
            out = lax.while_loop(w_cond, w_body, init)
            cs = out[2:]
            for v in range(n_vecs):
                seg = jnp.minimum(s0 + cs[v], NUM_BAGS - 1)
                half, off = divmod(v, SCAT // 16)
                seg_refs[half][pl.ds(off * 16, 16)] = seg

        def stage(g, b, fire_next, wait_old):
            bn = (b + 1) % NBUF
            wait_gather(g, b)
            if wait_old:
                wait_scatter(bn)
            if fire_next:
                fire_gather(g + 1, bn)
            compute_segs(g, bufs[b][0])
            fire_scatter(b)

        stage(0, 0, True, False)
        stage(1, 1, True, False)

        def triple(i, _):
            g = 3 * i + 2
            stage(g, 2, True, True)
            stage(g + 1, 0, True, True)
            stage(g + 2, 1, True, True)
            return _

        lax.fori_loop(0, (N_CHUNKS - 4) // 3, triple, None)
        stage(N_CHUNKS - 2, (N_CHUNKS - 2) % NBUF, True, True)
        stage(N_CHUNKS - 1, (N_CHUNKS - 1) % NBUF, False, True)
        wait_scatter((N_CHUNKS - 2) % NBUF)
        wait_scatter((N_CHUNKS - 1) % NBUF)
        plsc.subcore_barrier()

        pltpu.sync_copy(
            acc_sh.at[pl.ds(s * rows_per_tile, rows_per_tile)],
            out_hbm.at[pl.ds(c * NUM_BAGS + s * rows_per_tile,
                             rows_per_tile)])

    return body


def _combine_body(a_ref, b_ref, o_ref):
    o_ref[...] = a_ref[...] + b_ref[...]


@jax.jit
def kernel(weight, indices_fwd, offsets):
    partials = _sc_partials()(weight, indices_fwd, offsets)
    return pl.pallas_call(
        _combine_body,
        out_shape=jax.ShapeDtypeStruct((NUM_BAGS, EMBED_DIM), jnp.float32),
    )(partials[:NUM_BAGS], partials[NUM_BAGS:])
